# Initial kernel scaffold; baseline (speedup 1.0000x reference)
import functools

import jax
import jax.numpy as jnp
from jax import lax
from jax.experimental import pallas as pl
from jax.experimental.pallas import tpu as pltpu
from jax.experimental.pallas import tpu_sc as plsc

N_TILE = 256
K_CHUNK = 2048


def _argmin_body(xt_ref, wneg_ref, xsq_ref, esq_ref, out_ref):
    xt = xt_ref[...]
    xsq = xsq_ref[...]
    K = wneg_ref.shape[1]
    nt = xt.shape[0]
    run_min = jnp.full((nt, 1), jnp.inf, jnp.float32)
    run_idx = jnp.zeros((nt, 1), jnp.int32)
    for c in range(K // K_CHUNK):
        w_c = wneg_ref[:, c * K_CHUNK:(c + 1) * K_CHUNK]
        esq_c = esq_ref[:, c * K_CHUNK:(c + 1) * K_CHUNK]
        cross = jnp.dot(xt, w_c, preferred_element_type=jnp.float32)
        t = (xsq + cross) + esq_c
        dist = jnp.sqrt(jnp.maximum(t, 1e-12))
        cmin = jnp.min(dist, axis=1, keepdims=True)
        iota = lax.broadcasted_iota(jnp.int32, dist.shape, 1)
        cidx = jnp.min(jnp.where(dist == cmin, iota, K), axis=1,
                       keepdims=True) + c * K_CHUNK
        upd = cmin < run_min
        run_idx = jnp.where(upd, cidx, run_idx)
        run_min = jnp.where(upd, cmin, run_min)
    out_ref[...] = run_idx


def _argmin_call(xt, wneg, xsq, esq, interpret=False):
    N, d = xt.shape
    K = wneg.shape[1]
    return pl.pallas_call(
        _argmin_body,
        grid=(N // N_TILE,),
        in_specs=[
            pl.BlockSpec((N_TILE, d), lambda i: (i, 0)),
            pl.BlockSpec((d, K), lambda i: (0, 0)),
            pl.BlockSpec((N_TILE, 1), lambda i: (i, 0)),
            pl.BlockSpec((1, K), lambda i: (0, 0)),
        ],
        out_specs=pl.BlockSpec((N_TILE, 1), lambda i: (i, 0)),
        out_shape=jax.ShapeDtypeStruct((N, 1), jnp.int32),
        interpret=interpret,
    )(xt, wneg, xsq, esq)


@functools.cache
def _make_sc_gather(V, D, B_):
    info = plsc.get_sparse_core_info()
    NW = info.num_cores * info.num_subcores
    b_per_w = B_ // NW
    chunk = 128
    n_chunks = b_per_w // chunk
    mesh = plsc.VectorSubcoreMesh(core_axis_name="c", subcore_axis_name="s")

    @functools.partial(
        pl.kernel, mesh=mesh,
        out_type=jax.ShapeDtypeStruct((B_, D), jnp.float32),
        scratch_types=[
            pltpu.VMEM((n_chunks, chunk), jnp.int32),
            pltpu.VMEM((b_per_w, D), jnp.float32),
            pltpu.SemaphoreType.DMA,
        ],
    )
    def gk(table_hbm, idx_hbm, out_hbm, idx_v, rows_v, sem):
        wid = lax.axis_index("s") * info.num_cores + lax.axis_index("c")
        base = wid * b_per_w
        pltpu.sync_copy(idx_hbm.at[pl.ds(wid * n_chunks, n_chunks)], idx_v)
        copies = []
        for j in range(n_chunks):
            copies.append(pltpu.async_copy(
                table_hbm.at[idx_v.at[j]],
                rows_v.at[pl.ds(j * chunk, chunk)], sem))
        for cp in copies:
            cp.wait()
        pltpu.sync_copy(rows_v, out_hbm.at[pl.ds(base, b_per_w)])

    return gk


def kernel(x, weight):
    B, d = x.shape[0], x.shape[1]
    spatial = x.shape[2:]
    K = weight.shape[1]
    xt = jnp.transpose(x, (0, 2, 3, 1)).reshape(-1, d)
    N = xt.shape[0]
    xsq = jnp.sum(xt * xt, axis=1, keepdims=True)
    esq = jnp.sum(weight * weight, axis=0, keepdims=True)
    wneg = -2.0 * weight
    amin = _argmin_call(xt, wneg, xsq, esq)
    amin_flat = amin.reshape(N)
    wt = weight.T
    idx2d = amin_flat.reshape(N // 128, 128)
    result_flat = _make_sc_gather(K, d, N)(wt, idx2d)
    result = jnp.transpose(result_flat.reshape(B, *spatial, d), (0, 3, 1, 2))
    return result, amin_flat.reshape(B, *spatial)


# speedup vs baseline: 1.1842x; 1.1842x over previous
import functools

import jax
import jax.numpy as jnp
from jax import lax
from jax.experimental import pallas as pl
from jax.experimental.pallas import tpu as pltpu
from jax.experimental.pallas import tpu_sc as plsc

N_TILE = 256
K_CHUNK = 4096


def _argmin_body(xt_ref, wneg_ref, xsq_ref, esq_ref, out_ref, min_ref):
    xt = xt_ref[...]
    xsq = xsq_ref[...]
    K = wneg_ref.shape[1]
    nt = xt.shape[0]
    run_min = jnp.full((nt, 1), jnp.inf, jnp.float32)
    run_idx = jnp.zeros((nt, 1), jnp.int32)
    for c in range(K // K_CHUNK):
        w_c = wneg_ref[:, c * K_CHUNK:(c + 1) * K_CHUNK]
        esq_c = esq_ref[:, c * K_CHUNK:(c + 1) * K_CHUNK]
        cross = jnp.dot(xt, w_c, preferred_element_type=jnp.float32)
        t = (xsq + cross) + esq_c
        m = jnp.maximum(t, 1e-12)
        dist = m * lax.rsqrt(m)
        cmin = jnp.min(dist, axis=1, keepdims=True)
        iota = lax.broadcasted_iota(jnp.int32, dist.shape, 1)
        cidx = jnp.min(jnp.where(dist == cmin, iota, K), axis=1,
                       keepdims=True) + c * K_CHUNK
        upd = (cmin < run_min) | ((cmin == run_min) & (cidx < run_idx))
        run_idx = jnp.where(upd, cidx, run_idx)
        run_min = jnp.where(upd, cmin, run_min)
        run_min = run_min.astype(jnp.bfloat16).astype(jnp.float32)
    out_ref[...] = run_idx
    min_ref[...] = run_min


def _argmin_call(xt, wneg, xsq, esq, interpret=False):
    N, d = xt.shape
    K = wneg.shape[1]
    idx, _ = pl.pallas_call(
        _argmin_body,
        grid=(N // N_TILE,),
        in_specs=[
            pl.BlockSpec((N_TILE, d), lambda i: (i, 0)),
            pl.BlockSpec((d, K), lambda i: (0, 0)),
            pl.BlockSpec((N_TILE, 1), lambda i: (i, 0)),
            pl.BlockSpec((1, K), lambda i: (0, 0)),
        ],
        out_specs=[pl.BlockSpec((N_TILE, 1), lambda i: (i, 0)),
                   pl.BlockSpec((N_TILE, 1), lambda i: (i, 0))],
        out_shape=[jax.ShapeDtypeStruct((N, 1), jnp.int32),
                   jax.ShapeDtypeStruct((N, 1), jnp.float32)],
        interpret=interpret,
    )(xt, wneg, xsq, esq)
    return idx


@functools.cache
def _make_sc_gather(V, D, B_):
    info = plsc.get_sparse_core_info()
    NW = info.num_cores * info.num_subcores
    b_per_w = B_ // NW
    chunk = 128
    n_chunks = b_per_w // chunk
    mesh = plsc.VectorSubcoreMesh(core_axis_name="c", subcore_axis_name="s")

    @functools.partial(
        pl.kernel, mesh=mesh,
        out_type=jax.ShapeDtypeStruct((B_, D), jnp.float32),
        scratch_types=[
            pltpu.VMEM((n_chunks, chunk), jnp.int32),
            pltpu.VMEM((b_per_w, D), jnp.float32),
            pltpu.SemaphoreType.DMA,
        ],
    )
    def gk(table_hbm, idx_hbm, out_hbm, idx_v, rows_v, sem):
        wid = lax.axis_index("s") * info.num_cores + lax.axis_index("c")
        base = wid * b_per_w
        pltpu.sync_copy(idx_hbm.at[pl.ds(wid * n_chunks, n_chunks)], idx_v)
        copies = []
        for j in range(n_chunks):
            copies.append(pltpu.async_copy(
                table_hbm.at[idx_v.at[j]],
                rows_v.at[pl.ds(j * chunk, chunk)], sem))
        for cp in copies:
            cp.wait()
        pltpu.sync_copy(rows_v, out_hbm.at[pl.ds(base, b_per_w)])

    return gk


def kernel(x, weight):
    B, d = x.shape[0], x.shape[1]
    spatial = x.shape[2:]
    K = weight.shape[1]
    xt = jnp.transpose(x, (0, 2, 3, 1)).reshape(-1, d)
    N = xt.shape[0]
    xsq = jnp.sum(xt * xt, axis=1, keepdims=True)
    esq = jnp.sum(weight * weight, axis=0, keepdims=True)
    wneg = -2.0 * weight
    amin = _argmin_call(xt, wneg, xsq, esq)
    amin_flat = amin.reshape(N)
    dp = max(128, ((d + 127) // 128) * 128)
    wt = jnp.pad(weight.T, ((0, 0), (0, dp - d)))
    idx2d = amin_flat.reshape(N // 128, 128)
    gathered = _make_sc_gather(K, dp, N)(wt, idx2d)
    result_flat = gathered[:, :d]
    result = jnp.transpose(result_flat.reshape(B, *spatial, d), (0, 3, 1, 2))
    return result, amin_flat.reshape(B, *spatial)


# speedup vs baseline: 1.2904x; 1.0897x over previous
import functools

import jax
import jax.numpy as jnp
from jax import lax
from jax.experimental import pallas as pl
from jax.experimental.pallas import tpu as pltpu
from jax.experimental.pallas import tpu_sc as plsc

N_TILE = 256
K_CHUNK = 4096


def _argmin_body(xt_ref, wneg_ref, xsq_ref, esq_ref, out_ref):
    xt = xt_ref[...]
    xsq = xsq_ref[...]
    K = wneg_ref.shape[1]
    nt = xt.shape[0]
    run_min = jnp.full((nt, 1), jnp.inf, jnp.float32)
    run_idx = jnp.zeros((nt, 1), jnp.float32)
    for c in range(K // K_CHUNK):
        w_c = wneg_ref[:, c * K_CHUNK:(c + 1) * K_CHUNK]
        esq_c = esq_ref[:, c * K_CHUNK:(c + 1) * K_CHUNK]
        cross = jnp.dot(xt, w_c, preferred_element_type=jnp.float32)
        t = (xsq + cross) + esq_c
        m = jnp.maximum(t, 1e-12)
        dist = m * lax.rsqrt(m)
        cmin = jnp.min(dist, axis=1, keepdims=True)
        iota = lax.broadcasted_iota(jnp.int32, dist.shape, 1).astype(jnp.float32)
        cidx = jnp.min(jnp.where(dist == cmin, iota, float(K)), axis=1,
                       keepdims=True) + float(c * K_CHUNK)
        upd = (cmin < run_min) | ((cmin == run_min) & (cidx < run_idx))
        run_idx = jnp.where(upd, cidx, run_idx)
        run_min = jnp.where(upd, cmin, run_min)
        run_min = run_min.astype(jnp.bfloat16).astype(jnp.float32)
    out_ref[...] = run_idx.astype(jnp.int32)


def _argmin_call(xt, wneg, xsq, esq, interpret=False):
    N, d = xt.shape
    K = wneg.shape[1]
    return pl.pallas_call(
        _argmin_body,
        grid=(N // N_TILE,),
        in_specs=[
            pl.BlockSpec((N_TILE, d), lambda i: (i, 0)),
            pl.BlockSpec((d, K), lambda i: (0, 0)),
            pl.BlockSpec((N_TILE, 1), lambda i: (i, 0)),
            pl.BlockSpec((1, K), lambda i: (0, 0)),
        ],
        out_specs=pl.BlockSpec((N_TILE, 1), lambda i: (i, 0)),
        out_shape=jax.ShapeDtypeStruct((N, 1), jnp.int32),
        interpret=interpret,
    )(xt, wneg, xsq, esq)


@functools.cache
def _make_sc_gather(V, D, B_):
    info = plsc.get_sparse_core_info()
    NW = info.num_cores * info.num_subcores
    b_per_w = B_ // NW
    chunk = 128
    n_chunks = b_per_w // chunk
    mesh = plsc.VectorSubcoreMesh(core_axis_name="c", subcore_axis_name="s")

    @functools.partial(
        pl.kernel, mesh=mesh,
        out_type=jax.ShapeDtypeStruct((B_, D), jnp.float32),
        scratch_types=[
            pltpu.VMEM((n_chunks, chunk), jnp.int32),
            pltpu.VMEM((b_per_w, D), jnp.float32),
            pltpu.SemaphoreType.DMA,
        ],
    )
    def gk(table_hbm, idx_hbm, out_hbm, idx_v, rows_v, sem):
        wid = lax.axis_index("s") * info.num_cores + lax.axis_index("c")
        base = wid * b_per_w
        pltpu.sync_copy(idx_hbm.at[pl.ds(wid * n_chunks, n_chunks)], idx_v)
        copies = []
        for j in range(n_chunks):
            copies.append(pltpu.async_copy(
                table_hbm.at[idx_v.at[j]],
                rows_v.at[pl.ds(j * chunk, chunk)], sem))
        for cp in copies:
            cp.wait()
        pltpu.sync_copy(rows_v, out_hbm.at[pl.ds(base, b_per_w)])

    return gk


def kernel(x, weight):
    B, d = x.shape[0], x.shape[1]
    spatial = x.shape[2:]
    K = weight.shape[1]
    xt = jnp.transpose(x, (0, 2, 3, 1)).reshape(-1, d)
    N = xt.shape[0]
    xsq = jnp.sum(xt * xt, axis=1, keepdims=True)
    esq = jnp.sum(weight * weight, axis=0, keepdims=True)
    wneg = -2.0 * weight
    amin = _argmin_call(xt, wneg, xsq, esq)
    amin_flat = amin.reshape(N)
    dp = max(128, ((d + 127) // 128) * 128)
    wt = jnp.pad(weight.T, ((0, 0), (0, dp - d)))
    idx2d = amin_flat.reshape(N // 128, 128)
    gathered = _make_sc_gather(K, dp, N)(wt, idx2d)
    result_flat = gathered[:, :d]
    result = jnp.transpose(result_flat.reshape(B, *spatial, d), (0, 3, 1, 2))
    return result, amin_flat.reshape(B, *spatial)
